# Initial kernel scaffold; baseline (speedup 1.0000x reference)
import jax
import jax.numpy as jnp
from jax.experimental import pallas as pl


def kernel(feat, edge_index, W, b):
    raise NotImplementedError("write your pallas kernel here")


# speedup vs baseline: 6.0527x; 6.0527x over previous
import jax
import jax.numpy as jnp
from jax import lax
from jax.experimental import pallas as pl
from jax.experimental.pallas import tpu as pltpu
from jax.experimental.pallas import tpu_sc as plsc

_N = 10000
_D = 128
_E = 320000
_NC = 2
_NS = 16
_NT = _NC * _NS
_EPT = _E // _NT
_K = 80
_NCHUNK = _EPT // _K
_NP = 10240
_RPT = _NP // _NS
_ZR = 128


def _hop_body(g_hbm, src_hbm, dst_hbm, out_hbm, src_b, dst_b, rows_b, zb,
              acc, sem):
    cid = lax.axis_index("c")
    sid = lax.axis_index("s")
    base = (cid * _NS + sid) * _EPT

    def fill_zeros(r, _):
        for c8 in range(_D // 16):
            zb[r, pl.ds(c8 * 16, 16)] = jnp.zeros((16,), jnp.float32)
        return 0

    lax.fori_loop(0, _ZR, fill_zeros, 0)
    for z in range(_RPT // _ZR):
        pltpu.sync_copy(zb, acc.at[pl.ds(sid * _RPT + z * _ZR, _ZR)])
    plsc.subcore_barrier()

    def chunk(i, _):
        pltpu.sync_copy(src_hbm.at[pl.ds(base + i * _K, _K)], src_b)
        pltpu.sync_copy(dst_hbm.at[pl.ds(base + i * _K, _K)], dst_b)
        pltpu.async_copy(g_hbm.at[src_b], rows_b, sem).wait()
        pltpu.sync_copy(rows_b, acc.at[dst_b], add=True)
        return 0

    lax.fori_loop(0, _NCHUNK, chunk, 0)
    plsc.subcore_barrier()
    pltpu.sync_copy(acc.at[pl.ds(sid * _RPT, _RPT)],
                    out_hbm.at[cid, pl.ds(sid * _RPT, _RPT)])


def _sc_hop(g, src, dst):
    mesh = plsc.VectorSubcoreMesh(core_axis_name="c", subcore_axis_name="s",
                                  num_cores=_NC, num_subcores=_NS)
    return pl.kernel(
        _hop_body,
        out_type=jax.ShapeDtypeStruct((_NC, _NP, _D), jnp.float32),
        mesh=mesh,
        scratch_types=[
            pltpu.VMEM((_K,), jnp.int32),
            pltpu.VMEM((_K,), jnp.int32),
            pltpu.VMEM((_K, _D), jnp.float32),
            pltpu.VMEM((_ZR, _D), jnp.float32),
            pltpu.VMEM_SHARED((_NP, _D), jnp.float32),
            pltpu.SemaphoreType.DMA,
        ],
    )(g, src, dst)


_BLK = 1000


def _tc_transform_body(feat_ref, w_ref, b_ref, degp_ref, out_ref):
    dp = degp_ref[...]
    deg = dp[0, :, 0] + dp[1, :, 0]
    nrm = lax.rsqrt(jnp.maximum(deg, 1.0))
    h = jnp.dot(feat_ref[...], w_ref[...],
                preferred_element_type=jnp.float32) + b_ref[...]
    out_ref[...] = h * nrm[:, None]


def _tc_transform(feat, w, b2, degp):
    return pl.pallas_call(
        _tc_transform_body,
        grid=(_N // _BLK,),
        in_specs=[
            pl.BlockSpec((_BLK, _D), lambda i: (i, 0)),
            pl.BlockSpec((_D, _D), lambda i: (0, 0)),
            pl.BlockSpec((1, _D), lambda i: (0, 0)),
            pl.BlockSpec((_NC, _BLK, _D), lambda i: (0, i, 0)),
        ],
        out_specs=pl.BlockSpec((_BLK, _D), lambda i: (i, 0)),
        out_shape=jax.ShapeDtypeStruct((_N, _D), jnp.float32),
    )(feat, w, b2, degp)


def _tc_scale_body(s_ref, degp_ref, out_ref, power):
    dp = degp_ref[...]
    deg = dp[0, :, 0] + dp[1, :, 0]
    nrm = lax.rsqrt(jnp.maximum(deg, 1.0))
    scale = nrm * nrm if power == 2 else nrm
    s = s_ref[0] + s_ref[1]
    out_ref[...] = s * scale[:, None]


def _tc_scale(s, degp, power):
    body = lambda a, b, o: _tc_scale_body(a, b, o, power)
    return pl.pallas_call(
        body,
        grid=(_N // _BLK,),
        in_specs=[
            pl.BlockSpec((_NC, _BLK, _D), lambda i: (0, i, 0)),
            pl.BlockSpec((_NC, _BLK, _D), lambda i: (0, i, 0)),
        ],
        out_specs=pl.BlockSpec((_BLK, _D), lambda i: (i, 0)),
        out_shape=jax.ShapeDtypeStruct((_N, _D), jnp.float32),
    )(s, degp)


def kernel(feat, edge_index, W, b):
    src = edge_index[0]
    dst = edge_index[1]
    b2 = b.reshape(1, _D)
    ones = jnp.ones((_N, _D), jnp.float32)
    degp = _sc_hop(ones, src, dst)
    g0 = _tc_transform(feat, W, b2, degp)
    s1 = _sc_hop(g0, src, dst)
    g1 = _tc_scale(s1, degp, 2)
    s2 = _sc_hop(g1, src, dst)
    return _tc_scale(s2, degp, 1)
